# baseline (speedup 1.0000x reference)
import functools

import jax
import jax.numpy as jnp
from jax import lax
from jax.experimental import pallas as pl
from jax.experimental.pallas import tpu as pltpu
from jax.experimental.pallas import tpu_sc as plsc

_NC = 2
_NS = 16


def _sc_spmm(embeds, dstp, srcp, wp):
    N, D = embeds.shape
    NW, K, C = dstp.shape
    RPT = 624
    RB = 48
    T = RPT // RB
    TAIL = N - RPT * _NS
    NR = 4
    NI = 8

    mesh = plsc.VectorSubcoreMesh(core_axis_name="c", subcore_axis_name="s")

    scratch = (
        [pltpu.VMEM((C, D), jnp.float32) for _ in range(NR)]
        + [pltpu.VMEM((C,), jnp.int32) for _ in range(NI)]
        + [pltpu.VMEM((C,), jnp.int32) for _ in range(NI)]
        + [pltpu.VMEM((C,), jnp.float32) for _ in range(NI)]
        + [pltpu.VMEM_SHARED((N, D), jnp.float32)]
        + [pltpu.SemaphoreType.DMA] * NR
        + [pltpu.SemaphoreType.DMA] * NR
        + [pltpu.SemaphoreType.DMA] * NI
    )

    @functools.partial(
        pl.kernel,
        mesh=mesh,
        out_type=jax.ShapeDtypeStruct((_NC, N, D), jnp.float32),
        scratch_types=scratch,
    )
    def spmm(embeds_hbm, dst_hbm, src_hbm, w_hbm, out_hbm, *scr):
        rows = list(scr[0:NR])
        dsts = list(scr[NR:NR + NI])
        srcs = list(scr[NR + NI:NR + 2 * NI])
        ws = list(scr[NR + 2 * NI:NR + 3 * NI])
        agg_sp = scr[NR + 3 * NI]
        gsem = list(scr[NR + 3 * NI + 1:NR + 3 * NI + 1 + NR])
        ssem = list(scr[NR + 3 * NI + 1 + NR:NR + 3 * NI + 1 + 2 * NR])
        isem = list(scr[NR + 3 * NI + 1 + 2 * NR:])

        cid = lax.axis_index("c")
        sid = lax.axis_index("s")
        wid = cid * _NS + sid
        row0 = sid * RPT

        def fetch_idx(k, h):
            pltpu.async_copy(dst_hbm.at[wid, k], dsts[h], isem[h])
            pltpu.async_copy(src_hbm.at[wid, k], srcs[h], isem[h])
            pltpu.async_copy(w_hbm.at[wid, k], ws[h], isem[h])

        def wait_idx(k, h):
            pltpu.make_async_copy(dst_hbm.at[wid, k], dsts[h], isem[h]).wait()
            pltpu.make_async_copy(src_hbm.at[wid, k], srcs[h], isem[h]).wait()
            pltpu.make_async_copy(w_hbm.at[wid, k], ws[h], isem[h]).wait()

        def zb(i, carry):
            for j in range(D // 16):
                rows[0][i, pl.ds(j * 16, 16)] = jnp.zeros((16,), jnp.float32)
            return carry
        lax.fori_loop(0, RB, zb, 0)
        zsrc = rows[0].at[pl.ds(0, RB)]
        for t in range(T):
            pltpu.sync_copy(zsrc, agg_sp.at[pl.ds(row0 + t * RB, RB)])

        @pl.when(sid == _NS - 1)
        def _zero_tail():
            pltpu.sync_copy(rows[0].at[pl.ds(0, TAIL)],
                            agg_sp.at[pl.ds(RPT * _NS, TAIL)])
        plsc.subcore_barrier()

        for j in range(6):
            fetch_idx(j, j)
        wait_idx(0, 0)
        pltpu.async_copy(embeds_hbm.at[srcs[0]], rows[0], gsem[0])
        wait_idx(1, 1)
        pltpu.async_copy(embeds_hbm.at[srcs[1]], rows[1], gsem[1])

        def step(k, pos):
            j = pos % NR
            h = pos % NI
            j2 = (pos + 2) % NR
            h2 = (pos + 2) % NI
            hm2 = (pos + NI - 2) % NI
            h6 = (pos + 6) % NI

            @pl.when(k >= 2)
            def _drain():
                pltpu.make_async_copy(
                    rows[j2], agg_sp.at[dsts[hm2]], ssem[j2]).wait()

            @pl.when(k + 2 < K)
            def _prefetch():
                wait_idx(k + 2, h2)
                pltpu.async_copy(embeds_hbm.at[srcs[h2]], rows[j2], gsem[j2])

            pltpu.make_async_copy(embeds_hbm.at[srcs[h]], rows[j],
                                  gsem[j]).wait()

            def scale(g, c2):
                wv = ws[h][pl.ds(g * 16, 16)]
                for l in range(16):
                    wi = wv[l]
                    i = g * 16 + l
                    for jj in range(D // 16):
                        s = pl.ds(jj * 16, 16)
                        rows[j][i, s] = rows[j][i, s] * wi
                return c2
            lax.fori_loop(0, C // 16, scale, 0)

            pltpu.async_copy(rows[j], agg_sp.at[dsts[h]], ssem[j], add=True)

            @pl.when(k + 6 < K)
            def _fetch_next_idx():
                fetch_idx(k + 6, h6)

        def chunk8(kk, carry):
            k = kk * 8
            for pos in range(8):
                step(k + pos, pos)
            return carry
        lax.fori_loop(0, K // 8, chunk8, 0)

        pltpu.make_async_copy(rows[(K - 2) % NR],
                              agg_sp.at[dsts[(K - 2) % NI]],
                              ssem[(K - 2) % NR]).wait()
        pltpu.make_async_copy(rows[(K - 1) % NR],
                              agg_sp.at[dsts[(K - 1) % NI]],
                              ssem[(K - 1) % NR]).wait()

        plsc.subcore_barrier()
        rbuf = rows[0].at[pl.ds(0, RB)]
        for t in range(T):
            pltpu.sync_copy(agg_sp.at[pl.ds(row0 + t * RB, RB)], rbuf)
            pltpu.sync_copy(rbuf, out_hbm.at[cid, pl.ds(row0 + t * RB, RB)])

        @pl.when(sid == _NS - 1)
        def _read_tail():
            pltpu.sync_copy(agg_sp.at[pl.ds(RPT * _NS, TAIL)],
                            rows[0].at[pl.ds(0, TAIL)])
            pltpu.sync_copy(rows[0].at[pl.ds(0, TAIL)],
                            out_hbm.at[cid, pl.ds(RPT * _NS, TAIL)])

    return spmm(embeds, dstp, srcp, wp)


def _tc_finish(partials, aw):
    N, D = partials.shape[1], partials.shape[2]

    def body(p_ref, a_ref, o_ref):
        agg = p_ref[0] + p_ref[1]
        aw_col = a_ref[...]
        scores = jnp.matmul(agg, aw_col)
        m = jnp.max(scores)
        e = jnp.exp(scores - m)
        att = e / jnp.sum(e)
        out = agg * att
        o_ref[...] = jnp.where(out >= 0, out, out * 0.2)

    return pl.pallas_call(
        body,
        out_shape=jax.ShapeDtypeStruct((N, D), jnp.float32),
    )(partials, aw)


def kernel(embeds, edge_index, edge_weight, att_weight):
    dst = edge_index[0]
    src = edge_index[1]
    E = edge_weight.shape[0]
    NW = _NC * _NS
    C = 80
    K = -(-E // (NW * C))
    K = -(-K // 8) * 8
    E2 = NW * K * C
    dstp = jnp.pad(dst, (0, E2 - E)).reshape(NW, K, C)
    srcp = jnp.pad(src, (0, E2 - E)).reshape(NW, K, C)
    wp = jnp.pad(edge_weight, (0, E2 - E)).reshape(NW, K, C)
    partials = _sc_spmm(embeds, dstp, srcp, wp)
    return _tc_finish(partials, att_weight)


# speedup vs baseline: 2.2993x; 2.2993x over previous
import functools

import jax
import jax.numpy as jnp
from jax import lax
from jax.experimental import pallas as pl
from jax.experimental.pallas import tpu as pltpu
from jax.experimental.pallas import tpu_sc as plsc

_NC = 2
_NS = 16


def _sc_spmm(embeds, pk):
    N, D = embeds.shape
    NW, K, _, C = pk.shape
    RPT = 624
    RB = 48
    T = RPT // RB
    TAIL = N - RPT * _NS

    mesh = plsc.VectorSubcoreMesh(core_axis_name="c", subcore_axis_name="s")

    @functools.partial(
        pl.kernel,
        mesh=mesh,
        out_type=jax.ShapeDtypeStruct((_NC, N, D), jnp.float32),
        scratch_types=[
            pltpu.VMEM((C, D), jnp.float32),
            pltpu.VMEM((C, D), jnp.float32),
            pltpu.VMEM((3, C), jnp.int32),
            pltpu.VMEM((3, C), jnp.int32),
            pltpu.VMEM_SHARED((N, D), jnp.float32),
            pltpu.SemaphoreType.DMA,
            pltpu.SemaphoreType.DMA,
        ],
    )
    def spmm(embeds_hbm, pk_hbm, out_hbm,
             rows_a, rows_b, pka, pkb, agg_sp, ga, gb):
        cid = lax.axis_index("c")
        sid = lax.axis_index("s")
        wid = cid * _NS + sid
        row0 = sid * RPT

        def zb(i, carry):
            for j in range(D // 16):
                rows_a[i, pl.ds(j * 16, 16)] = jnp.zeros((16,), jnp.float32)
            return carry
        lax.fori_loop(0, RB, zb, 0)
        zsrc = rows_a.at[pl.ds(0, RB)]
        for t in range(T):
            pltpu.sync_copy(zsrc, agg_sp.at[pl.ds(row0 + t * RB, RB)])

        @pl.when(sid == _NS - 1)
        def _zero_tail():
            pltpu.sync_copy(rows_a.at[pl.ds(0, TAIL)],
                            agg_sp.at[pl.ds(RPT * _NS, TAIL)])
        plsc.subcore_barrier()

        pltpu.sync_copy(pk_hbm.at[wid, 0], pka)
        pltpu.async_copy(embeds_hbm.at[pka.at[1]], rows_a, ga)

        def compute(k, rows, pk_cur, gsem):
            pltpu.make_async_copy(embeds_hbm.at[pk_cur.at[1]], rows,
                                  gsem).wait()

            def scale(g, c2):
                wv = lax.bitcast_convert_type(
                    pk_cur[2, pl.ds(g * 16, 16)], jnp.float32)
                for l in range(16):
                    wi = wv[l]
                    i = g * 16 + l
                    for j in range(D // 16):
                        s = pl.ds(j * 16, 16)
                        rows[i, s] = rows[i, s] * wi
                return c2
            lax.fori_loop(0, C // 16, scale, 0)
            pltpu.sync_copy(rows, agg_sp.at[pk_cur.at[0]], add=True)

        def step(k, rows, pk_cur, gsem, pk_nxt, rows_nxt, gsem_nxt):
            pltpu.sync_copy(pk_hbm.at[wid, k + 1], pk_nxt)
            pltpu.async_copy(embeds_hbm.at[pk_nxt.at[1]], rows_nxt, gsem_nxt)
            compute(k, rows, pk_cur, gsem)

        def chunk2(kk, carry):
            k = kk * 2
            step(k, rows_a, pka, ga, pkb, rows_b, gb)
            step(k + 1, rows_b, pkb, gb, pka, rows_a, ga)
            return carry
        lax.fori_loop(0, (K - 1) // 2, chunk2, 0)
        lastb = (K - 1) % 2
        compute(K - 1, rows_b if lastb else rows_a,
                pkb if lastb else pka, gb if lastb else ga)

        plsc.subcore_barrier()
        rbuf = rows_a.at[pl.ds(0, RB)]
        for t in range(T):
            pltpu.sync_copy(agg_sp.at[pl.ds(row0 + t * RB, RB)], rbuf)
            pltpu.sync_copy(rbuf, out_hbm.at[cid, pl.ds(row0 + t * RB, RB)])

        @pl.when(sid == _NS - 1)
        def _read_tail():
            pltpu.sync_copy(agg_sp.at[pl.ds(RPT * _NS, TAIL)],
                            rows_a.at[pl.ds(0, TAIL)])
            pltpu.sync_copy(rows_a.at[pl.ds(0, TAIL)],
                            out_hbm.at[cid, pl.ds(RPT * _NS, TAIL)])

    return spmm(embeds, pk)


def _tc_finish(partials, aw):
    N, D = partials.shape[1], partials.shape[2]

    def body(p_ref, a_ref, o_ref):
        agg = p_ref[0] + p_ref[1]
        aw_col = a_ref[...]
        scores = jnp.matmul(agg, aw_col)
        m = jnp.max(scores)
        e = jnp.exp(scores - m)
        att = e / jnp.sum(e)
        out = agg * att
        o_ref[...] = jnp.where(out >= 0, out, out * 0.2)

    return pl.pallas_call(
        body,
        out_shape=jax.ShapeDtypeStruct((N, D), jnp.float32),
    )(partials, aw)


def kernel(embeds, edge_index, edge_weight, att_weight):
    dst = edge_index[0]
    src = edge_index[1]
    E = edge_weight.shape[0]
    NW = _NC * _NS
    C = 80
    K = E // (NW * C)
    wbits = lax.bitcast_convert_type(edge_weight, jnp.int32)
    pk = jnp.stack([dst.reshape(NW, K, C), src.reshape(NW, K, C),
                    wbits.reshape(NW, K, C)], axis=2)
    partials = _sc_spmm(embeds, pk)
    return _tc_finish(partials, att_weight)


# speedup vs baseline: 2.6811x; 1.1660x over previous
import functools

import jax
import jax.numpy as jnp
from jax import lax
from jax.experimental import pallas as pl
from jax.experimental.pallas import tpu as pltpu
from jax.experimental.pallas import tpu_sc as plsc

_NC = 2
_NS = 16


def _sc_spmm(embeds, pk):
    N, D = embeds.shape
    NW, K, _, C = pk.shape
    RPT = 624
    RB = 48
    T = RPT // RB
    TAIL = N - RPT * _NS
    NSTEADY = ((K - 5) // 3) * 3

    mesh = plsc.VectorSubcoreMesh(core_axis_name="c", subcore_axis_name="s")

    @functools.partial(
        pl.kernel,
        mesh=mesh,
        out_type=jax.ShapeDtypeStruct((_NC, N, D), jnp.float32),
        scratch_types=[
            pltpu.VMEM((C, D), jnp.float32),
            pltpu.VMEM((C, D), jnp.float32),
            pltpu.VMEM((C, D), jnp.float32),
            pltpu.VMEM((3, C), jnp.int32),
            pltpu.VMEM((3, C), jnp.int32),
            pltpu.VMEM((3, C), jnp.int32),
            pltpu.VMEM_SHARED((N, D), jnp.float32),
            pltpu.SemaphoreType.DMA,
            pltpu.SemaphoreType.DMA,
            pltpu.SemaphoreType.DMA,
            pltpu.SemaphoreType.DMA,
            pltpu.SemaphoreType.DMA,
            pltpu.SemaphoreType.DMA,
        ],
    )
    def spmm(embeds_hbm, pk_hbm, out_hbm,
             r0, r1, r2, pv0, pv1, pv2, agg_sp, g0, g1, g2, s0, s1, s2):
        rows = (r0, r1, r2)
        pkv = (pv0, pv1, pv2)
        gsem = (g0, g1, g2)
        ssem = (s0, s1, s2)

        cid = lax.axis_index("c")
        sid = lax.axis_index("s")
        wid = cid * _NS + sid
        row0 = sid * RPT

        def zb(i, carry):
            for j in range(D // 16):
                rows[0][i, pl.ds(j * 16, 16)] = jnp.zeros((16,), jnp.float32)
            return carry
        lax.fori_loop(0, RB, zb, 0)
        zsrc = rows[0].at[pl.ds(0, RB)]
        for t in range(T):
            pltpu.sync_copy(zsrc, agg_sp.at[pl.ds(row0 + t * RB, RB)])

        @pl.when(sid == _NS - 1)
        def _zero_tail():
            pltpu.sync_copy(rows[0].at[pl.ds(0, TAIL)],
                            agg_sp.at[pl.ds(RPT * _NS, TAIL)])
        plsc.subcore_barrier()

        def fetch(k, p):
            pltpu.sync_copy(pk_hbm.at[wid, k], pkv[p])
            pltpu.async_copy(embeds_hbm.at[pkv[p].at[1]], rows[p], gsem[p])

        def drain(p):
            pltpu.make_async_copy(rows[p], agg_sp.at[pkv[p].at[0]],
                                  ssem[p]).wait()

        def compute(k, p):
            pltpu.make_async_copy(embeds_hbm.at[pkv[p].at[1]], rows[p],
                                  gsem[p]).wait()

            def scale(g, c2):
                wv = lax.bitcast_convert_type(
                    pkv[p][2, pl.ds(g * 16, 16)], jnp.float32)
                for l in range(16):
                    wi = wv[l]
                    i = g * 16 + l
                    for j in range(D // 16):
                        s = pl.ds(j * 16, 16)
                        rows[p][i, s] = rows[p][i, s] * wi
                return c2
            lax.fori_loop(0, C // 16, scale, 0)
            pltpu.async_copy(rows[p], agg_sp.at[pkv[p].at[0]], ssem[p],
                             add=True)

        fetch(0, 0)
        fetch(1, 1)
        compute(0, 0)
        fetch(2, 2)
        compute(1, 1)

        def chunk3(kk, carry):
            for u in range(3):
                k = 2 + kk * 3 + u
                p = (2 + u) % 3
                p1 = (2 + u + 1) % 3
                drain(p1)
                fetch(k + 1, p1)
                compute(k, p)
            return carry
        lax.fori_loop(0, NSTEADY // 3, chunk3, 0)

        for k in range(2 + NSTEADY, K):
            p = k % 3
            p1 = (k + 1) % 3
            drain(p1)
            if k + 1 < K:
                fetch(k + 1, p1)
            compute(k, p)
        drain((K - 2) % 3)
        drain((K - 1) % 3)

        plsc.subcore_barrier()
        rbuf = rows[0].at[pl.ds(0, RB)]
        for t in range(T):
            pltpu.sync_copy(agg_sp.at[pl.ds(row0 + t * RB, RB)], rbuf)
            pltpu.sync_copy(rbuf, out_hbm.at[cid, pl.ds(row0 + t * RB, RB)])

        @pl.when(sid == _NS - 1)
        def _read_tail():
            pltpu.sync_copy(agg_sp.at[pl.ds(RPT * _NS, TAIL)],
                            rows[0].at[pl.ds(0, TAIL)])
            pltpu.sync_copy(rows[0].at[pl.ds(0, TAIL)],
                            out_hbm.at[cid, pl.ds(RPT * _NS, TAIL)])

    return spmm(embeds, pk)


def _tc_finish(partials, aw):
    N, D = partials.shape[1], partials.shape[2]

    def body(p_ref, a_ref, o_ref):
        agg = p_ref[0] + p_ref[1]
        aw_col = a_ref[...]
        scores = jnp.matmul(agg, aw_col)
        m = jnp.max(scores)
        e = jnp.exp(scores - m)
        att = e / jnp.sum(e)
        out = agg * att
        o_ref[...] = jnp.where(out >= 0, out, out * 0.2)

    return pl.pallas_call(
        body,
        out_shape=jax.ShapeDtypeStruct((N, D), jnp.float32),
    )(partials, aw)


def kernel(embeds, edge_index, edge_weight, att_weight):
    dst = edge_index[0]
    src = edge_index[1]
    E = edge_weight.shape[0]
    NW = _NC * _NS
    C = 80
    K = E // (NW * C)
    wbits = lax.bitcast_convert_type(edge_weight, jnp.int32)
    pk = jnp.stack([dst.reshape(NW, K, C), src.reshape(NW, K, C),
                    wbits.reshape(NW, K, C)], axis=2)
    partials = _sc_spmm(embeds, pk)
    return _tc_finish(partials, att_weight)


# speedup vs baseline: 2.7506x; 1.0259x over previous
import functools

import jax
import jax.numpy as jnp
from jax import lax
from jax.experimental import pallas as pl
from jax.experimental.pallas import tpu as pltpu
from jax.experimental.pallas import tpu_sc as plsc

_NC = 2
_NS = 16


def _sc_spmm(embeds, pk):
    N, D = embeds.shape
    NW, K, _, C = pk.shape
    RPT = 624
    RB = 48
    T = RPT // RB
    TAIL = N - RPT * _NS
    NSTEADY = ((K - 7) // 4) * 4

    mesh = plsc.VectorSubcoreMesh(core_axis_name="c", subcore_axis_name="s")

    @functools.partial(
        pl.kernel,
        mesh=mesh,
        out_type=jax.ShapeDtypeStruct((_NC, N, D), jnp.float32),
        scratch_types=[
            pltpu.VMEM((C, D), jnp.float32),
            pltpu.VMEM((C, D), jnp.float32),
            pltpu.VMEM((C, D), jnp.float32),
            pltpu.VMEM((C, D), jnp.float32),
            pltpu.VMEM((3, C), jnp.int32),
            pltpu.VMEM((3, C), jnp.int32),
            pltpu.VMEM((3, C), jnp.int32),
            pltpu.VMEM((3, C), jnp.int32),
            pltpu.VMEM_SHARED((N, D), jnp.float32),
            pltpu.SemaphoreType.DMA,
            pltpu.SemaphoreType.DMA,
            pltpu.SemaphoreType.DMA,
            pltpu.SemaphoreType.DMA,
            pltpu.SemaphoreType.DMA,
            pltpu.SemaphoreType.DMA,
            pltpu.SemaphoreType.DMA,
            pltpu.SemaphoreType.DMA,
        ],
    )
    def spmm(embeds_hbm, pk_hbm, out_hbm,
             r0, r1, r2, r3, pv0, pv1, pv2, pv3, agg_sp,
             g0, g1, g2, g3, s0, s1, s2, s3):
        rows = (r0, r1, r2, r3)
        pkv = (pv0, pv1, pv2, pv3)
        gsem = (g0, g1, g2, g3)
        ssem = (s0, s1, s2, s3)

        cid = lax.axis_index("c")
        sid = lax.axis_index("s")
        wid = cid * _NS + sid
        row0 = sid * RPT

        def zb(i, carry):
            for j in range(D // 16):
                rows[0][i, pl.ds(j * 16, 16)] = jnp.zeros((16,), jnp.float32)
            return carry
        lax.fori_loop(0, RB, zb, 0)
        zsrc = rows[0].at[pl.ds(0, RB)]
        for t in range(T):
            pltpu.sync_copy(zsrc, agg_sp.at[pl.ds(row0 + t * RB, RB)])

        @pl.when(sid == _NS - 1)
        def _zero_tail():
            pltpu.sync_copy(rows[0].at[pl.ds(0, TAIL)],
                            agg_sp.at[pl.ds(RPT * _NS, TAIL)])
        plsc.subcore_barrier()

        def fetch(k, p):
            pltpu.sync_copy(pk_hbm.at[wid, k], pkv[p])
            pltpu.async_copy(embeds_hbm.at[pkv[p].at[1]], rows[p], gsem[p])

        def drain(p):
            pltpu.make_async_copy(rows[p], agg_sp.at[pkv[p].at[0]],
                                  ssem[p]).wait()

        def compute(k, p):
            pltpu.make_async_copy(embeds_hbm.at[pkv[p].at[1]], rows[p],
                                  gsem[p]).wait()

            def scale(g, c2):
                wv = lax.bitcast_convert_type(
                    pkv[p][2, pl.ds(g * 16, 16)], jnp.float32)
                for l in range(16):
                    wi = wv[l]
                    i = g * 16 + l
                    for j in range(D // 16):
                        s = pl.ds(j * 16, 16)
                        rows[p][i, s] = rows[p][i, s] * wi
                return c2
            lax.fori_loop(0, C // 16, scale, 0)
            pltpu.async_copy(rows[p], agg_sp.at[pkv[p].at[0]], ssem[p],
                             add=True)

        fetch(0, 0)
        fetch(1, 1)
        fetch(2, 2)
        compute(0, 0)
        fetch(3, 3)
        compute(1, 1)

        def chunk4(kk, carry):
            for u in range(4):
                k = 2 + kk * 4 + u
                p = (2 + u) % 4
                p2 = (2 + u + 2) % 4
                drain(p2)
                fetch(k + 2, p2)
                compute(k, p)
            return carry
        lax.fori_loop(0, NSTEADY // 4, chunk4, 0)

        for k in range(2 + NSTEADY, K):
            p = k % 4
            p2 = (k + 2) % 4
            if k + 2 < K + 2:
                drain(p2)
            if k + 2 < K:
                fetch(k + 2, p2)
            compute(k, p)
        drain((K - 2) % 4)
        drain((K - 1) % 4)

        plsc.subcore_barrier()
        rbuf = rows[0].at[pl.ds(0, RB)]
        for t in range(T):
            pltpu.sync_copy(agg_sp.at[pl.ds(row0 + t * RB, RB)], rbuf)
            pltpu.sync_copy(rbuf, out_hbm.at[cid, pl.ds(row0 + t * RB, RB)])

        @pl.when(sid == _NS - 1)
        def _read_tail():
            pltpu.sync_copy(agg_sp.at[pl.ds(RPT * _NS, TAIL)],
                            rows[0].at[pl.ds(0, TAIL)])
            pltpu.sync_copy(rows[0].at[pl.ds(0, TAIL)],
                            out_hbm.at[cid, pl.ds(RPT * _NS, TAIL)])

    return spmm(embeds, pk)


def _tc_finish(partials, aw):
    N, D = partials.shape[1], partials.shape[2]

    def body(p_ref, a_ref, o_ref):
        agg = p_ref[0] + p_ref[1]
        aw_col = a_ref[...]
        scores = jnp.matmul(agg, aw_col)
        m = jnp.max(scores)
        e = jnp.exp(scores - m)
        att = e / jnp.sum(e)
        out = agg * att
        o_ref[...] = jnp.where(out >= 0, out, out * 0.2)

    return pl.pallas_call(
        body,
        out_shape=jax.ShapeDtypeStruct((N, D), jnp.float32),
    )(partials, aw)


def kernel(embeds, edge_index, edge_weight, att_weight):
    dst = edge_index[0]
    src = edge_index[1]
    E = edge_weight.shape[0]
    NW = _NC * _NS
    C = 80
    K = E // (NW * C)
    wbits = lax.bitcast_convert_type(edge_weight, jnp.int32)
    pk = jnp.stack([dst.reshape(NW, K, C), src.reshape(NW, K, C),
                    wbits.reshape(NW, K, C)], axis=2)
    partials = _sc_spmm(embeds, pk)
    return _tc_finish(partials, att_weight)


# speedup vs baseline: 2.9662x; 1.0784x over previous
import functools

import jax
import jax.numpy as jnp
from jax import lax
from jax.experimental import pallas as pl
from jax.experimental.pallas import tpu as pltpu
from jax.experimental.pallas import tpu_sc as plsc

_NC = 2
_NS = 16


def _sc_spmm(embeds, pk):
    N, D = embeds.shape
    NW, K, _, C = pk.shape
    RPT = 624
    RB = 48
    T = RPT // RB
    TAIL = N - RPT * _NS
    NSTEADY = ((K - 7) // 4) * 4

    mesh = plsc.VectorSubcoreMesh(core_axis_name="c", subcore_axis_name="s")

    @functools.partial(
        pl.kernel,
        mesh=mesh,
        out_type=jax.ShapeDtypeStruct((_NC, N, D), jnp.float32),
        scratch_types=[
            pltpu.VMEM((C, D), jnp.float32),
            pltpu.VMEM((C, D), jnp.float32),
            pltpu.VMEM((C, D), jnp.float32),
            pltpu.VMEM((C, D), jnp.float32),
            pltpu.VMEM((3, C), jnp.int32),
            pltpu.VMEM((3, C), jnp.int32),
            pltpu.VMEM((3, C), jnp.int32),
            pltpu.VMEM((3, C), jnp.int32),
            pltpu.VMEM_SHARED((N, D), jnp.float32),
            pltpu.SemaphoreType.DMA,
            pltpu.SemaphoreType.DMA,
            pltpu.SemaphoreType.DMA,
            pltpu.SemaphoreType.DMA,
            pltpu.SemaphoreType.DMA,
            pltpu.SemaphoreType.DMA,
            pltpu.SemaphoreType.DMA,
            pltpu.SemaphoreType.DMA,
        ],
    )
    def spmm(embeds_hbm, pk_hbm, out_hbm,
             r0, r1, r2, r3, pv0, pv1, pv2, pv3, agg_sp,
             g0, g1, g2, g3, s0, s1, s2, s3):
        rows = (r0, r1, r2, r3)
        pkv = (pv0, pv1, pv2, pv3)
        gsem = (g0, g1, g2, g3)
        ssem = (s0, s1, s2, s3)

        cid = lax.axis_index("c")
        sid = lax.axis_index("s")
        wid = cid * _NS + sid
        row0 = sid * RPT

        def zb(i, carry):
            for j in range(D // 16):
                rows[0][i, pl.ds(j * 16, 16)] = jnp.zeros((16,), jnp.float32)
            return carry
        lax.fori_loop(0, RB, zb, 0)
        zsrc = rows[0].at[pl.ds(0, RB)]
        for t in range(T):
            pltpu.sync_copy(zsrc, agg_sp.at[pl.ds(row0 + t * RB, RB)])

        @pl.when(sid == _NS - 1)
        def _zero_tail():
            pltpu.sync_copy(rows[0].at[pl.ds(0, TAIL)],
                            agg_sp.at[pl.ds(RPT * _NS, TAIL)])
        plsc.subcore_barrier()

        def fetch(k, p):
            pltpu.sync_copy(pk_hbm.at[wid, k], pkv[p])
            pltpu.async_copy(embeds_hbm.at[pkv[p].at[1]], rows[p], gsem[p])

        def drain(p):
            pltpu.make_async_copy(rows[p], agg_sp.at[pkv[p].at[0]],
                                  ssem[p]).wait()

        def compute(k, p):
            pltpu.make_async_copy(embeds_hbm.at[pkv[p].at[1]], rows[p],
                                  gsem[p]).wait()

            def scale(g, c2):
                wv = lax.bitcast_convert_type(
                    pkv[p][2, pl.ds(g * 16, 16)], jnp.float32)
                for l in range(16):
                    wi = wv[l]
                    i = g * 16 + l
                    for j in range(D // 16):
                        s = pl.ds(j * 16, 16)
                        rows[p][i, s] = rows[p][i, s] * wi
                return c2
            lax.fori_loop(0, C // 16, scale, 0)
            pltpu.async_copy(rows[p], agg_sp.at[pkv[p].at[0]], ssem[p],
                             add=True)

        fetch(0, 0)
        fetch(1, 1)
        fetch(2, 2)
        compute(0, 0)
        fetch(3, 3)
        compute(1, 1)

        def chunk4(kk, carry):
            for u in range(4):
                k = 2 + kk * 4 + u
                p = (2 + u) % 4
                p2 = (2 + u + 2) % 4
                drain(p2)
                fetch(k + 2, p2)
                compute(k, p)
            return carry
        lax.fori_loop(0, NSTEADY // 4, chunk4, 0)

        for k in range(2 + NSTEADY, K):
            p = k % 4
            p2 = (k + 2) % 4
            if k + 2 < K + 2:
                drain(p2)
            if k + 2 < K:
                fetch(k + 2, p2)
            compute(k, p)
        drain((K - 2) % 4)
        drain((K - 1) % 4)

        plsc.subcore_barrier()
        rbuf = rows[0].at[pl.ds(0, RB)]
        for t in range(T):
            pltpu.sync_copy(agg_sp.at[pl.ds(row0 + t * RB, RB)], rbuf)
            pltpu.sync_copy(rbuf, out_hbm.at[cid, pl.ds(row0 + t * RB, RB)])

        @pl.when(sid == _NS - 1)
        def _read_tail():
            pltpu.sync_copy(agg_sp.at[pl.ds(RPT * _NS, TAIL)],
                            rows[0].at[pl.ds(0, TAIL)])
            pltpu.sync_copy(rows[0].at[pl.ds(0, TAIL)],
                            out_hbm.at[cid, pl.ds(RPT * _NS, TAIL)])

    return spmm(embeds, pk)


def _tc_finish(partials, aw):
    N, D = partials.shape[1], partials.shape[2]

    def body(p_ref, a_ref, o_ref):
        agg = p_ref[0] + p_ref[1]
        aw_col = a_ref[...]
        scores = jnp.matmul(agg, aw_col)
        m = jnp.max(scores)
        e = jnp.exp(scores - m)
        att = e / jnp.sum(e)
        out = agg * att
        o_ref[...] = jnp.where(out >= 0, out, out * 0.2)

    return pl.pallas_call(
        body,
        out_shape=jax.ShapeDtypeStruct((N, D), jnp.float32),
    )(partials, aw)


def kernel(embeds, edge_index, edge_weight, att_weight):
    E = edge_weight.shape[0]
    NW = _NC * _NS
    C = 80
    K = E // (NW * C)
    wbits = lax.bitcast_convert_type(edge_weight, jnp.int32)
    eiw = jnp.concatenate([edge_index, wbits[None]], axis=0)
    pk = jnp.transpose(eiw.reshape(3, NW, K, C), (1, 2, 0, 3))
    partials = _sc_spmm(embeds, pk)
    return _tc_finish(partials, att_weight)
